# baseline (speedup 1.0000x reference)
import functools

import jax
import jax.numpy as jnp
import numpy as np
from jax import lax
from jax.experimental import pallas as pl
from jax.experimental.pallas import tpu as pltpu
from jax.experimental.pallas import tpu_sc as plsc

_SIZE = 100000
_N_ROWS = 1024
_SMOOTHING = 0.1
_CONF = 1.0 - _SMOOTHING
_SVAL = _SMOOTHING / (_SIZE - 2)

_S32 = np.float32(_SVAL)
_ENT = float(
    (_SIZE - 2) * np.float64(np.float32(_S32 * np.float32(np.log(_S32))))
    + np.float64(np.float32(np.float32(_CONF) * np.float32(np.log(np.float32(_CONF)))))
)

_NC = 2
_NS = 16
_NW = _NC * _NS
_CR = 32
_CSPLIT = 49984
_TC_BC = 4168
_TC_NBLK = (_SIZE - _CSPLIT) // _TC_BC
_NCHUNK = _CSPLIT // _CR
_BASE_K = _NCHUNK // _NW
_EXTRA = _NCHUNK % _NW
_NBUF = 3
_NACC = 8
_NGRP = _N_ROWS // 16
_NFLAG = 128


def _sc_body(xt_hbm, tgt_hbm, out_hbm,
             b0, b1, b2, tgt_v, mask_v, flag_v, tot_v, gt_v, g0_v, res_v,
             s0, s1, s2):
    bufs = (b0, b1, b2)
    sems = (s0, s1, s2)
    cid = lax.axis_index("c")
    sid = lax.axis_index("s")
    wid = sid * _NC + cid
    nk = _BASE_K + jnp.where(wid < _EXTRA, 1, 0)

    pltpu.sync_copy(tgt_hbm.at[pl.ds(0, _N_ROWS)], tgt_v)

    zero16 = jnp.zeros((16,), jnp.float32)
    zero16i = jnp.zeros((16,), jnp.int32)
    one16i = jnp.ones((16,), jnp.int32)
    tot_v[...] = zero16
    gt_v[...] = zero16
    g0_v[...] = zero16

    iota16 = lax.iota(jnp.int32, 16)
    widv = jnp.full((16,), wid, jnp.int32)

    for i in range(_NFLAG // 16):
        flag_v[pl.ds(i * 16, 16)] = zero16i

    def prol(g, c):
        tv = tgt_v[pl.ds(g * 16, 16)]
        nz = tv != 0
        mask_v[pl.ds(g * 16, 16)] = jnp.where(nz, 1.0, 0.0)
        ch = lax.shift_right_logical(tv, 5)
        mine = nz & (tv < _CSPLIT) & ((ch & 31) == widv)
        lc = lax.shift_right_logical(ch, 5)
        plsc.store_scatter(flag_v, [lc], one16i, mask=mine)
        return c
    lax.fori_loop(0, _NGRP, prol, 0)

    def start_chunk(k, buf, sem):
        ci = wid + k * _NW
        pltpu.make_async_copy(xt_hbm.at[pl.ds(ci * _CR, _CR)], buf, sem).start()

    for b in range(_NBUF):
        start_chunk(b, bufs[b], sems[b])

    def chunk_body(k, c):
        b = k % _NBUF
        ci = wid + k * _NW

        def with_buf(buf, sem):
            pltpu.make_async_copy(
                xt_hbm.at[pl.ds(0, _CR)], buf, sem).wait()

            def inner(bb, accs, buf=buf):
                mv = mask_v[pl.ds(bb, 16)]
                out = list(accs)
                for cls in range(_CR):
                    out[cls % _NACC] = (
                        out[cls % _NACC] + buf[cls, pl.ds(bb, 16)] * mv)
                return tuple(out)
            accs = plsc.parallel_loop(
                0, _N_ROWS, 16, unroll=2,
                carry=(zero16,) * _NACC)(inner)
            vs = list(accs)
            while len(vs) > 1:
                nxt = [vs[p] + vs[p + 1] for p in range(0, len(vs) - 1, 2)]
                if len(vs) % 2:
                    nxt.append(vs[-1])
                vs = nxt
            tot_v[...] = tot_v[...] + vs[0]

            grp = flag_v[pl.ds((k >> 4) << 4, 16)]
            hit = jnp.any((grp != 0) & (iota16 == (k & 15)))

            @pl.when(hit)
            def _(buf=buf):
                civ = jnp.full((16,), ci, jnp.int32)

                def scan(g, c2, buf=buf):
                    tv = tgt_v[pl.ds(g * 16, 16)]
                    sel = (tv != 0) & (
                        lax.shift_right_logical(tv, 5) == civ)
                    cls = tv & (_CR - 1)
                    bidx = g * 16 + iota16
                    gv = plsc.load_gather(buf, [cls, bidx])
                    gt_v[...] = gt_v[...] + jnp.where(sel, gv, 0.0)
                    return c2
                lax.fori_loop(0, _NGRP, scan, 0)

            @pl.when(ci == 0)
            def _(buf=buf):
                def col0(g, c2, buf=buf):
                    g0_v[...] = (g0_v[...]
                                 + buf[0, pl.ds(g * 16, 16)]
                                 * mask_v[pl.ds(g * 16, 16)])
                    return c2
                lax.fori_loop(0, _NGRP, col0, 0)

            kn = k + _NBUF

            @pl.when(kn < nk)
            def _(kn=kn, buf=buf, sem=sem):
                start_chunk(kn, buf, sem)

        for bb in range(_NBUF):
            @pl.when(b == bb)
            def _(bb=bb):
                with_buf(bufs[bb], sems[bb])
        return c

    lax.fori_loop(0, nk, chunk_body, 0)

    cnt = zero16

    def cnt_loop(g, c):
        return c + mask_v[pl.ds(g * 16, 16)]
    cnt = lax.fori_loop(0, _NGRP, cnt_loop, cnt)
    w0 = widv == 0
    cnt = jnp.where(w0, cnt, 0.0)

    sval = jnp.float32(_SVAL)
    res = (cnt * jnp.float32(_ENT)
           - sval * tot_v[...]
           + sval * g0_v[...]
           + jnp.float32(_SVAL - _CONF) * gt_v[...])
    res_v[...] = res
    pltpu.sync_copy(res_v, out_hbm.at[pl.ds(wid * 16, 16)])


def _tc_body(xt_ref, tgt_ref, out_ref):
    i = pl.program_id(0)

    @pl.when(i == 0)
    def _():
        out_ref[...] = jnp.zeros((1, 2), jnp.float32)

    xb = xt_ref[...]
    tg = tgt_ref[...]
    m = jnp.where(tg != 0, 1.0, 0.0).astype(jnp.float32)
    s_part = jnp.sum(xb * m)
    cls = (lax.broadcasted_iota(jnp.int32, (_TC_BC, _N_ROWS), 0)
           + _CSPLIT + i * _TC_BC)
    g_part = jnp.sum(jnp.where(cls == tg, xb, 0.0))
    out_ref[...] = out_ref[...] + jnp.concatenate(
        [s_part.reshape(1, 1), g_part.reshape(1, 1)], axis=1)


@jax.jit
def kernel(x, target):
    tgt = target.astype(jnp.int32)
    xt = x.T
    mesh = plsc.VectorSubcoreMesh(core_axis_name="c", subcore_axis_name="s")
    f = functools.partial(
        pl.kernel,
        mesh=mesh,
        compiler_params=pltpu.CompilerParams(needs_layout_passes=False),
        out_type=jax.ShapeDtypeStruct((_NW * 16,), jnp.float32),
        scratch_types=[
            pltpu.VMEM((_CR, _N_ROWS), jnp.float32),
            pltpu.VMEM((_CR, _N_ROWS), jnp.float32),
            pltpu.VMEM((_CR, _N_ROWS), jnp.float32),
            pltpu.VMEM((_N_ROWS,), jnp.int32),
            pltpu.VMEM((_N_ROWS,), jnp.float32),
            pltpu.VMEM((_NFLAG,), jnp.int32),
            pltpu.VMEM((16,), jnp.float32),
            pltpu.VMEM((16,), jnp.float32),
            pltpu.VMEM((16,), jnp.float32),
            pltpu.VMEM((16,), jnp.float32),
            pltpu.SemaphoreType.DMA,
            pltpu.SemaphoreType.DMA,
            pltpu.SemaphoreType.DMA,
        ],
    )(_sc_body)
    partials = f(xt, tgt)

    tcres = pl.pallas_call(
        _tc_body,
        grid=(_TC_NBLK,),
        out_shape=jax.ShapeDtypeStruct((1, 2), jnp.float32),
        in_specs=[
            pl.BlockSpec((_TC_BC, _N_ROWS),
                         lambda i: (_CSPLIT // _TC_BC + i, 0)),
            pl.BlockSpec((1, _N_ROWS), lambda i: (0, 0)),
        ],
        out_specs=pl.BlockSpec((1, 2), lambda i: (0, 0)),
    )(xt, tgt.reshape(1, _N_ROWS))

    return (jnp.sum(partials)
            - jnp.float32(_SVAL) * tcres[0, 0]
            + jnp.float32(_SVAL - _CONF) * tcres[0, 1])


# speedup vs baseline: 1.0046x; 1.0046x over previous
import functools

import jax
import jax.numpy as jnp
import numpy as np
from jax import lax
from jax.experimental import pallas as pl
from jax.experimental.pallas import tpu as pltpu
from jax.experimental.pallas import tpu_sc as plsc

_SIZE = 100000
_N_ROWS = 1024
_SMOOTHING = 0.1
_CONF = 1.0 - _SMOOTHING
_SVAL = _SMOOTHING / (_SIZE - 2)

_S32 = np.float32(_SVAL)
_ENT = float(
    (_SIZE - 2) * np.float64(np.float32(_S32 * np.float32(np.log(_S32))))
    + np.float64(np.float32(np.float32(_CONF) * np.float32(np.log(np.float32(_CONF)))))
)

_NC = 2
_NS = 16
_NW = _NC * _NS
_CR = 32
_CSPLIT = 48000
_TC_BC = 2000
_TC_NBLK = (_SIZE - _CSPLIT) // _TC_BC
_NCHUNK = _CSPLIT // _CR
_BASE_K = _NCHUNK // _NW
_EXTRA = _NCHUNK % _NW
_NBUF = 3
_NACC = 8
_NGRP = _N_ROWS // 16
_NFLAG = 128


def _sc_body(xt_hbm, tgt_hbm, out_hbm,
             b0, b1, b2, tgt_v, mask_v, flag_v, tot_v, gt_v, g0_v, res_v,
             s0, s1, s2):
    bufs = (b0, b1, b2)
    sems = (s0, s1, s2)
    cid = lax.axis_index("c")
    sid = lax.axis_index("s")
    wid = sid * _NC + cid
    nk = _BASE_K + jnp.where(wid < _EXTRA, 1, 0)

    pltpu.sync_copy(tgt_hbm.at[pl.ds(0, _N_ROWS)], tgt_v)

    zero16 = jnp.zeros((16,), jnp.float32)
    zero16i = jnp.zeros((16,), jnp.int32)
    one16i = jnp.ones((16,), jnp.int32)
    tot_v[...] = zero16
    gt_v[...] = zero16
    g0_v[...] = zero16

    iota16 = lax.iota(jnp.int32, 16)
    widv = jnp.full((16,), wid, jnp.int32)

    for i in range(_NFLAG // 16):
        flag_v[pl.ds(i * 16, 16)] = zero16i

    def prol(g, c):
        tv = tgt_v[pl.ds(g * 16, 16)]
        nz = tv != 0
        mask_v[pl.ds(g * 16, 16)] = jnp.where(nz, 1.0, 0.0)
        ch = lax.shift_right_logical(tv, 5)
        mine = nz & (tv < _CSPLIT) & ((ch & 31) == widv)
        lc = lax.shift_right_logical(ch, 5)
        plsc.store_scatter(flag_v, [lc], one16i, mask=mine)
        return c
    lax.fori_loop(0, _NGRP, prol, 0)

    def start_chunk(k, buf, sem):
        ci = wid + k * _NW
        pltpu.make_async_copy(xt_hbm.at[pl.ds(ci * _CR, _CR)], buf, sem).start()

    for b in range(_NBUF):
        start_chunk(b, bufs[b], sems[b])

    def chunk_body(k, c):
        b = k % _NBUF
        ci = wid + k * _NW

        def with_buf(buf, sem):
            pltpu.make_async_copy(
                xt_hbm.at[pl.ds(0, _CR)], buf, sem).wait()

            def inner(bb, accs, buf=buf):
                mv = mask_v[pl.ds(bb, 16)]
                out = list(accs)
                for cls in range(_CR):
                    out[cls % _NACC] = (
                        out[cls % _NACC] + buf[cls, pl.ds(bb, 16)] * mv)
                return tuple(out)
            accs = plsc.parallel_loop(
                0, _N_ROWS, 16, unroll=2,
                carry=(zero16,) * _NACC)(inner)
            vs = list(accs)
            while len(vs) > 1:
                nxt = [vs[p] + vs[p + 1] for p in range(0, len(vs) - 1, 2)]
                if len(vs) % 2:
                    nxt.append(vs[-1])
                vs = nxt
            tot_v[...] = tot_v[...] + vs[0]

            grp = flag_v[pl.ds((k >> 4) << 4, 16)]
            hit = jnp.any((grp != 0) & (iota16 == (k & 15)))

            @pl.when(hit)
            def _(buf=buf):
                civ = jnp.full((16,), ci, jnp.int32)

                def scan(g, c2, buf=buf):
                    tv = tgt_v[pl.ds(g * 16, 16)]
                    sel = (tv != 0) & (
                        lax.shift_right_logical(tv, 5) == civ)
                    cls = tv & (_CR - 1)
                    bidx = g * 16 + iota16
                    gv = plsc.load_gather(buf, [cls, bidx])
                    gt_v[...] = gt_v[...] + jnp.where(sel, gv, 0.0)
                    return c2
                lax.fori_loop(0, _NGRP, scan, 0)

            @pl.when(ci == 0)
            def _(buf=buf):
                def col0(g, c2, buf=buf):
                    g0_v[...] = (g0_v[...]
                                 + buf[0, pl.ds(g * 16, 16)]
                                 * mask_v[pl.ds(g * 16, 16)])
                    return c2
                lax.fori_loop(0, _NGRP, col0, 0)

            kn = k + _NBUF

            @pl.when(kn < nk)
            def _(kn=kn, buf=buf, sem=sem):
                start_chunk(kn, buf, sem)

        for bb in range(_NBUF):
            @pl.when(b == bb)
            def _(bb=bb):
                with_buf(bufs[bb], sems[bb])
        return c

    lax.fori_loop(0, nk, chunk_body, 0)

    cnt = zero16

    def cnt_loop(g, c):
        return c + mask_v[pl.ds(g * 16, 16)]
    cnt = lax.fori_loop(0, _NGRP, cnt_loop, cnt)
    w0 = widv == 0
    cnt = jnp.where(w0, cnt, 0.0)

    sval = jnp.float32(_SVAL)
    res = (cnt * jnp.float32(_ENT)
           - sval * tot_v[...]
           + sval * g0_v[...]
           + jnp.float32(_SVAL - _CONF) * gt_v[...])
    res_v[...] = res
    pltpu.sync_copy(res_v, out_hbm.at[pl.ds(wid * 16, 16)])


def _tc_body(xt_ref, tgt_ref, out_ref):
    i = pl.program_id(0)

    @pl.when(i == 0)
    def _():
        out_ref[...] = jnp.zeros((1, 2), jnp.float32)

    xb = xt_ref[...]
    tg = tgt_ref[...]
    m = jnp.where(tg != 0, 1.0, 0.0).astype(jnp.float32)
    s_part = jnp.sum(xb * m)
    cls = (lax.broadcasted_iota(jnp.int32, (_TC_BC, _N_ROWS), 0)
           + _CSPLIT + i * _TC_BC)
    g_part = jnp.sum(jnp.where(cls == tg, xb, 0.0))
    out_ref[...] = out_ref[...] + jnp.concatenate(
        [s_part.reshape(1, 1), g_part.reshape(1, 1)], axis=1)


@jax.jit
def kernel(x, target):
    tgt = target.astype(jnp.int32)
    xt = x.T
    mesh = plsc.VectorSubcoreMesh(core_axis_name="c", subcore_axis_name="s")
    f = functools.partial(
        pl.kernel,
        mesh=mesh,
        compiler_params=pltpu.CompilerParams(needs_layout_passes=False),
        out_type=jax.ShapeDtypeStruct((_NW * 16,), jnp.float32),
        scratch_types=[
            pltpu.VMEM((_CR, _N_ROWS), jnp.float32),
            pltpu.VMEM((_CR, _N_ROWS), jnp.float32),
            pltpu.VMEM((_CR, _N_ROWS), jnp.float32),
            pltpu.VMEM((_N_ROWS,), jnp.int32),
            pltpu.VMEM((_N_ROWS,), jnp.float32),
            pltpu.VMEM((_NFLAG,), jnp.int32),
            pltpu.VMEM((16,), jnp.float32),
            pltpu.VMEM((16,), jnp.float32),
            pltpu.VMEM((16,), jnp.float32),
            pltpu.VMEM((16,), jnp.float32),
            pltpu.SemaphoreType.DMA,
            pltpu.SemaphoreType.DMA,
            pltpu.SemaphoreType.DMA,
        ],
    )(_sc_body)
    partials = f(xt, tgt)

    tcres = pl.pallas_call(
        _tc_body,
        grid=(_TC_NBLK,),
        out_shape=jax.ShapeDtypeStruct((1, 2), jnp.float32),
        in_specs=[
            pl.BlockSpec((_TC_BC, _N_ROWS),
                         lambda i: (_CSPLIT // _TC_BC + i, 0)),
            pl.BlockSpec((1, _N_ROWS), lambda i: (0, 0)),
        ],
        out_specs=pl.BlockSpec((1, 2), lambda i: (0, 0)),
    )(xt, tgt.reshape(1, _N_ROWS))

    return (jnp.sum(partials)
            - jnp.float32(_SVAL) * tcres[0, 0]
            + jnp.float32(_SVAL - _CONF) * tcres[0, 1])
